# Initial kernel scaffold; baseline (speedup 1.0000x reference)
import jax
import jax.numpy as jnp
from jax.experimental import pallas as pl


def kernel(q, r, A, B_pair, lengths, pair_index, relation_features, Wq1, bq1, Wq2, bq2, We, be, Wu, W1, b1, W2, b2, W3, b3):
    raise NotImplementedError("write your pallas kernel here")


# speedup vs baseline: 2.7336x; 2.7336x over previous
import functools

import jax
import jax.numpy as jnp
from jax.experimental import pallas as pl

B = 512
D_Q = 1024
D_Z = 512
D_RIN = 256
D_PHI = 128
D_R = 9
M = 512
P = 256
NPROG = 8192
LAM = 0.1
H = 96

TB = 128
TPROG = 1024


def _dotnn(a, b):
    return jax.lax.dot_general(a, b, (((1,), (0,)), ((), ())),
                               preferred_element_type=jnp.float32)


def _dotnt(a, b):
    return jax.lax.dot_general(a, b, (((1,), (1,)), ((), ())),
                               preferred_element_type=jnp.float32)


def _stage1_kernel(q_ref, r_ref, pair_idx_ref, relf_ref,
                   Wq1_ref, bq1_ref, Wq2_ref, bq2_ref,
                   We_ref, be_ref, Wu_ref,
                   W1z_ref, W1s_ref, W1a_ref, W1m_ref, W1r_ref, b1_ref,
                   W2_ref, b2_ref, W3_ref, b3_ref,
                   u_ref, v_ref):
    qg = jax.nn.gelu(_dotnn(q_ref[...], Wq1_ref[...]) + bq1_ref[...])
    g = _dotnn(qg, Wq2_ref[...]) + bq2_ref[...]

    phi = _dotnn(r_ref[...], We_ref[...]) + be_ref[...]

    u_ref[...] = _dotnt(_dotnn(g, Wu_ref[...]), phi)

    i_col = pair_idx_ref[:, 0:1]
    j_col = pair_idx_ref[:, 1:2]
    iota = jax.lax.broadcasted_iota(jnp.int32, (P, M), 1)
    oh_i = (i_col == iota).astype(jnp.float32)
    oh_j = (j_col == iota).astype(jnp.float32)
    phi_i = _dotnn(oh_i, phi)
    phi_j = _dotnn(oh_j, phi)
    sym_sum = phi_i + phi_j
    sym_abs = jnp.abs(phi_i - phi_j)
    sym_prod = phi_i * phi_j

    az = _dotnn(g, W1z_ref[...]) + b1_ref[...]
    ap = (_dotnn(sym_sum, W1s_ref[...]) + _dotnn(sym_abs, W1a_ref[...])
          + _dotnn(sym_prod, W1m_ref[...]) + _dotnn(relf_ref[...], W1r_ref[...]))

    h1 = jax.nn.gelu(az[:, None, :] + ap[None, :, :])
    h1 = h1.reshape(TB * P, H)
    h2 = jax.nn.gelu(_dotnn(h1, W2_ref[...]) + b2_ref[...])
    v = _dotnn(h2, W3_ref[...]) + b3_ref[...]
    v_ref[...] = v.reshape(TB, P)


def _stage2_kernel(u_ref, v_ref, A_ref, Bp_ref, len_ref, out_ref):
    s = _dotnt(u_ref[...], A_ref[...]) + _dotnt(v_ref[...], Bp_ref[...])
    out_ref[...] = s - LAM * len_ref[...]


@jax.jit
def kernel(q, r, A, B_pair, lengths, pair_index, relation_features,
           Wq1, bq1, Wq2, bq2, We, be, Wu, W1, b1, W2, b2, W3, b3):
    f32 = jnp.float32
    W1z = W1[:D_Z]
    W1s = W1[D_Z:D_Z + D_PHI]
    W1a = W1[D_Z + D_PHI:D_Z + 2 * D_PHI]
    W1m = W1[D_Z + 2 * D_PHI:D_Z + 3 * D_PHI]
    W1r = W1[D_Z + 3 * D_PHI:]
    bq1_2 = bq1.reshape(1, -1)
    bq2_2 = bq2.reshape(1, -1)
    be_2 = be.reshape(1, -1)
    b1_2 = b1.reshape(1, -1)
    b2_2 = b2.reshape(1, -1)
    b3_2 = b3.reshape(1, -1)
    pair_idx = pair_index.astype(jnp.int32)
    len_2 = lengths.reshape(1, NPROG)

    nbt = B // TB
    rep = lambda shape: pl.BlockSpec(shape, lambda b: (0,) * len(shape))
    u, v = pl.pallas_call(
        _stage1_kernel,
        grid=(nbt,),
        in_specs=[
            pl.BlockSpec((TB, D_Q), lambda b: (b, 0)),
            rep((M, D_RIN)),
            rep((P, 2)),
            rep((P, D_R)),
            rep((D_Q, 512)), rep((1, 512)),
            rep((512, D_Z)), rep((1, D_Z)),
            rep((D_RIN, D_PHI)), rep((1, D_PHI)),
            rep((D_Z, D_PHI)),
            rep((D_Z, H)), rep((D_PHI, H)), rep((D_PHI, H)), rep((D_PHI, H)),
            rep((D_R, H)), rep((1, H)),
            rep((H, H)), rep((1, H)),
            rep((H, 1)), rep((1, 1)),
        ],
        out_specs=[
            pl.BlockSpec((TB, M), lambda b: (b, 0)),
            pl.BlockSpec((TB, P), lambda b: (b, 0)),
        ],
        out_shape=[
            jax.ShapeDtypeStruct((B, M), f32),
            jax.ShapeDtypeStruct((B, P), f32),
        ],
    )(q, r, pair_idx, relation_features,
      Wq1, bq1_2, Wq2, bq2_2, We, be_2, Wu,
      W1z, W1s, W1a, W1m, W1r, b1_2, W2, b2_2, W3, b3_2)

    scores = pl.pallas_call(
        _stage2_kernel,
        grid=(NPROG // TPROG,),
        in_specs=[
            pl.BlockSpec((B, M), lambda p: (0, 0)),
            pl.BlockSpec((B, P), lambda p: (0, 0)),
            pl.BlockSpec((TPROG, M), lambda p: (p, 0)),
            pl.BlockSpec((TPROG, P), lambda p: (p, 0)),
            pl.BlockSpec((1, TPROG), lambda p: (0, p)),
        ],
        out_specs=pl.BlockSpec((B, TPROG), lambda p: (0, p)),
        out_shape=jax.ShapeDtypeStruct((B, NPROG), f32),
    )(u, v, A, B_pair, len_2)
    return scores


# speedup vs baseline: 3.9681x; 1.4516x over previous
import functools

import jax
import jax.numpy as jnp
from jax.experimental import pallas as pl
from jax.experimental.pallas import tpu as pltpu

B = 512
D_Q = 1024
D_Z = 512
D_RIN = 256
D_PHI = 128
D_R = 9
M = 512
P = 256
NPROG = 8192
LAM = 0.1
H = 96

TB = 128
TPROG = 1024
NBUF = 6
NT = NPROG // TPROG


def _gelu_erf(x):
    halfx = x * jnp.asarray(0.5, x.dtype)
    return halfx * jax.lax.erf(x * jnp.asarray(0.7071067811865476, x.dtype)) + halfx


def _dotnn(a, b):
    return jax.lax.dot_general(a, b, (((1,), (0,)), ((), ())),
                               preferred_element_type=jnp.float32)


def _dotnt(a, b):
    return jax.lax.dot_general(a, b, (((1,), (1,)), ((), ())),
                               preferred_element_type=jnp.float32)


def _router_kernel(q_ref, r_ref, pair_idx_ref, relf_ref,
                   Wq1_ref, bq1_ref, Wq2_ref, bq2_ref,
                   We_ref, be_ref, Wu_ref,
                   W1z_ref, W1s_ref, W1a_ref, W1m_ref, W1r_ref, b1_ref,
                   W2_ref, b2_ref, W3_ref, b3_ref, len_ref,
                   A_hbm, Bp_hbm,
                   out_hbm,
                   u_s, v_s, A_buf, Bp_buf, out_buf, in_sem, out_sem):
    bf = jnp.bfloat16

    def a_copy(t):
        slot = t % NBUF
        return pltpu.make_async_copy(
            A_hbm.at[pl.ds(t * TPROG, TPROG), :], A_buf.at[slot], in_sem.at[slot])

    def b_copy(t):
        slot = t % NBUF
        return pltpu.make_async_copy(
            Bp_hbm.at[pl.ds(t * TPROG, TPROG), :], Bp_buf.at[slot], in_sem.at[slot])

    def o_copy(t):
        oslot = t % 2
        return pltpu.make_async_copy(
            out_buf.at[oslot], out_hbm.at[:, pl.ds(t * TPROG, TPROG)], out_sem.at[oslot])

    for t in range(min(NBUF, NT)):
        a_copy(t).start()
        b_copy(t).start()

    phi = _dotnn(r_ref[...], We_ref[...]) + be_ref[...]
    i_col = pair_idx_ref[:, 0:1]
    j_col = pair_idx_ref[:, 1:2]
    iota = jax.lax.broadcasted_iota(jnp.int32, (P, M), 1)
    oh_i = (i_col == iota).astype(jnp.float32)
    oh_j = (j_col == iota).astype(jnp.float32)
    phi_i = _dotnn(oh_i, phi)
    phi_j = _dotnn(oh_j, phi)
    sym_sum = phi_i + phi_j
    sym_abs = jnp.abs(phi_i - phi_j)
    sym_prod = phi_i * phi_j
    ap = (_dotnn(sym_sum, W1s_ref[...]) + _dotnn(sym_abs, W1a_ref[...])
          + _dotnn(sym_prod, W1m_ref[...]) + _dotnn(relf_ref[...], W1r_ref[...]))
    ap_bf = ap.astype(bf)

    for bt in range(B // TB):
        sl = pl.ds(bt * TB, TB)
        qg = jax.nn.gelu(_dotnn(q_ref[sl, :], Wq1_ref[...]) + bq1_ref[...])
        g = _dotnn(qg, Wq2_ref[...]) + bq2_ref[...]
        u_s[sl, :] = _dotnt(_dotnn(g, Wu_ref[...]), phi)
        az = _dotnn(g, W1z_ref[...]) + b1_ref[...]
        az_bf = az.astype(bf)
        h1 = _gelu_erf(az_bf[:, None, :] + ap_bf[None, :, :])
        h1 = h1.reshape(TB * P, H)
        h2pre = _dotnn(h1, W2_ref[...].astype(bf)) + b2_ref[...]
        h2 = _gelu_erf(h2pre.astype(bf))
        v = _dotnn(h2, W3_ref[...].astype(bf)) + b3_ref[...]
        v_s[sl, :] = v.reshape(TB, P)

    u_bf = u_s[...].astype(bf)
    v_bf = v_s[...].astype(bf)

    for t in range(NT):
        slot = t % NBUF
        a_copy(t).wait()
        b_copy(t).wait()
        s = (_dotnt(u_bf, A_buf[slot].astype(bf))
             + _dotnt(v_bf, Bp_buf[slot].astype(bf)))
        s = s - LAM * len_ref[:, pl.ds(t * TPROG, TPROG)]
        oslot = t % 2
        if t >= 2:
            o_copy(t - 2).wait()
        out_buf[oslot] = s
        o_copy(t).start()
        if t + NBUF < NT:
            a_copy(t + NBUF).start()
            b_copy(t + NBUF).start()
    for t in range(max(NT - 2, 0), NT):
        o_copy(t).wait()


@jax.jit
def kernel(q, r, A, B_pair, lengths, pair_index, relation_features,
           Wq1, bq1, Wq2, bq2, We, be, Wu, W1, b1, W2, b2, W3, b3):
    f32 = jnp.float32
    W1z = W1[:D_Z]
    W1s = W1[D_Z:D_Z + D_PHI]
    W1a = W1[D_Z + D_PHI:D_Z + 2 * D_PHI]
    W1m = W1[D_Z + 2 * D_PHI:D_Z + 3 * D_PHI]
    W1r = W1[D_Z + 3 * D_PHI:]
    bq1_2 = bq1.reshape(1, -1)
    bq2_2 = bq2.reshape(1, -1)
    be_2 = be.reshape(1, -1)
    b1_2 = b1.reshape(1, -1)
    b2_2 = b2.reshape(1, -1)
    b3_2 = b3.reshape(1, -1)
    pair_idx = pair_index.astype(jnp.int32)
    len_2 = lengths.reshape(1, NPROG)

    vmem = lambda: pl.BlockSpec(memory_space=pltpu.MemorySpace.VMEM)
    hbm = lambda: pl.BlockSpec(memory_space=pltpu.MemorySpace.HBM)
    scores = pl.pallas_call(
        _router_kernel,
        in_specs=[vmem() for _ in range(22)] + [hbm(), hbm()],
        out_specs=pl.BlockSpec(memory_space=pltpu.MemorySpace.HBM),
        out_shape=jax.ShapeDtypeStruct((B, NPROG), f32),
        scratch_shapes=[
            pltpu.VMEM((B, M), f32),
            pltpu.VMEM((B, P), f32),
            pltpu.VMEM((NBUF, TPROG, M), f32),
            pltpu.VMEM((NBUF, TPROG, P), f32),
            pltpu.VMEM((2, B, TPROG), f32),
            pltpu.SemaphoreType.DMA((NBUF,)),
            pltpu.SemaphoreType.DMA((2,)),
        ],
    )(q, r, pair_idx, relation_features,
      Wq1, bq1_2, Wq2, bq2_2, We, be_2, Wu,
      W1z, W1s, W1a, W1m, W1r, b1_2, W2, b2_2, W3, b3_2, len_2,
      A, B_pair)
    return scores
